# baseline (speedup 1.0000x reference)
import functools

import jax
import jax.numpy as jnp
from jax import lax
from jax.experimental import pallas as pl
from jax.experimental.pallas import tpu as pltpu
from jax.experimental.pallas import tpu_sc as plsc

N = 10000
E = 320000
D = 128

NC = 2
NS = 16
NW = NC * NS
EPW = E // NW
K = 80
NCHUNK = EPW // K
R0 = 624
R15 = N - 15 * R0
DEGW = 128

_MESH = plsc.VectorSubcoreMesh(core_axis_name="c", subcore_axis_name="s")


@functools.partial(
    pl.kernel, mesh=_MESH,
    out_type=[jax.ShapeDtypeStruct((NC, N, D), jnp.float32)],
    scratch_types=[
        pltpu.VMEM_SHARED((N, D), jnp.float32),
        pltpu.VMEM((EPW,), jnp.int32),
        pltpu.VMEM((EPW,), jnp.int32),
        pltpu.VMEM((3, K, D), jnp.float32),
        pltpu.SemaphoreType.DMA,
        pltpu.SemaphoreType.DMA,
        pltpu.SemaphoreType.DMA,
        pltpu.SemaphoreType.DMA,
        pltpu.SemaphoreType.DMA,
        pltpu.SemaphoreType.DMA,
    ])
def _sc_agg(x_hbm, src_hbm, dst_hbm, out_hbm, acc_sh, sidx, didx, rows,
            sem0, sem1, sem2, ssem0, ssem1, ssem2):
    cid = lax.axis_index("c")
    sid = lax.axis_index("s")
    wid = cid * NS + sid
    row0 = sid * R0
    last = sid == NS - 1
    sems = (sem0, sem1, sem2)
    ssems = (ssem0, ssem1, ssem2)

    def _gather(j, b):
        return pltpu.make_async_copy(
            x_hbm.at[sidx.at[pl.ds(j * K, K)]], rows.at[b], sems[b])

    def _scatter(j, b):
        return pltpu.make_async_copy(
            rows.at[b], acc_sh.at[didx.at[pl.ds(j * K, K)]], ssems[b])

    pltpu.sync_copy(src_hbm.at[pl.ds(wid * EPW, EPW)], sidx)
    pltpu.sync_copy(dst_hbm.at[pl.ds(wid * EPW, EPW)], didx)

    def zrow(r, carry):
        for j in range(D // 16):
            rows[0, r, pl.ds(j * 16, 16)] = jnp.zeros((16,), jnp.float32)
        return carry
    lax.fori_loop(0, K, zrow, 0)

    @pl.when(jnp.logical_and(cid == 0, jnp.logical_not(last)))
    def _():
        pltpu.sync_copy(x_hbm.at[pl.ds(row0, R0)],
                        acc_sh.at[pl.ds(row0, R0)])

    @pl.when(jnp.logical_and(cid == 0, last))
    def _():
        pltpu.sync_copy(x_hbm.at[pl.ds(row0, R15)],
                        acc_sh.at[pl.ds(row0, R15)])

    @pl.when(jnp.logical_and(cid != 0, jnp.logical_not(last)))
    def _():
        for t in range(7):
            pltpu.sync_copy(rows.at[0],
                            acc_sh.at[pl.ds(row0 + t * K, K)])
        pltpu.sync_copy(rows.at[0].at[pl.ds(0, 64)],
                        acc_sh.at[pl.ds(row0 + 7 * K, 64)])

    @pl.when(jnp.logical_and(cid != 0, last))
    def _():
        for t in range(8):
            pltpu.sync_copy(rows.at[0],
                            acc_sh.at[pl.ds(row0 + t * K, K)])

    plsc.subcore_barrier()

    _gather(0, 0).start()
    _gather(1, 1).start()

    def body(t, carry):
        for o in range(3):
            j = 3 * t + o
            bn = (o + 2) % 3
            _gather(j, o).wait()

            @pl.when(j > 0)
            def _():
                _scatter(j - 1, bn).wait()
            _gather(j + 2, bn).start()
            _scatter(j, o).start(add=True)
        return carry
    lax.fori_loop(0, (NCHUNK - 2) // 3, body, 0)
    _gather(NCHUNK - 2, 0).wait()
    _scatter(NCHUNK - 3, 2).wait()
    _scatter(NCHUNK - 2, 0).start(add=True)
    _gather(NCHUNK - 1, 1).wait()
    _scatter(NCHUNK - 1, 1).start(add=True)
    _scatter(NCHUNK - 2, 0).wait()
    _scatter(NCHUNK - 1, 1).wait()
    plsc.subcore_barrier()

    @pl.when(jnp.logical_not(last))
    def _():
        pltpu.sync_copy(acc_sh.at[pl.ds(row0, R0)],
                        out_hbm.at[cid, pl.ds(row0, R0)])

    @pl.when(last)
    def _():
        pltpu.sync_copy(acc_sh.at[pl.ds(row0, R15)],
                        out_hbm.at[cid, pl.ds(row0, R15)])


@functools.partial(
    pl.kernel, mesh=_MESH,
    out_type=[jax.ShapeDtypeStruct((NC, N, DEGW), jnp.float32)],
    scratch_types=[
        pltpu.VMEM_SHARED((N, DEGW), jnp.float32),
        pltpu.VMEM((NCHUNK, K), jnp.int32),
        pltpu.VMEM((K, DEGW), jnp.float32),
        pltpu.VMEM((16, DEGW), jnp.float32),
        pltpu.SemaphoreType.DMA,
    ])
def _sc_deg(dst_hbm, degout_hbm, deg_sh, didx, ones, zdbuf, sem):
    cid = lax.axis_index("c")
    sid = lax.axis_index("s")
    wid = cid * NS + sid
    row0 = sid * R0
    last = sid == NS - 1

    def orow(r, carry):
        for j in range(DEGW // 16):
            ones[r, pl.ds(j * 16, 16)] = jnp.full((16,), 1.0, jnp.float32)
        return carry
    lax.fori_loop(0, K, orow, 0)

    def zrow(r, carry):
        for j in range(DEGW // 16):
            zdbuf[r, pl.ds(j * 16, 16)] = jnp.zeros((16,), jnp.float32)
        return carry
    lax.fori_loop(0, 16, zrow, 0)

    @pl.when(jnp.logical_not(last))
    def _():
        for t in range(R0 // 16):
            pltpu.sync_copy(zdbuf, deg_sh.at[pl.ds(row0 + t * 16, 16)])

    @pl.when(last)
    def _():
        for t in range(R15 // 16):
            pltpu.sync_copy(zdbuf, deg_sh.at[pl.ds(row0 + t * 16, 16)])

    pltpu.sync_copy(dst_hbm.at[wid], didx)
    plsc.subcore_barrier()

    def fire(j, carry):
        pltpu.async_copy(ones, deg_sh.at[didx.at[j]], sem, add=True)
        return carry
    lax.fori_loop(0, NCHUNK, fire, 0)

    def drain(j, carry):
        pltpu.make_async_copy(ones, deg_sh.at[didx.at[j]], sem).wait()
        return carry
    lax.fori_loop(0, NCHUNK, drain, 0)
    plsc.subcore_barrier()

    @pl.when(jnp.logical_not(last))
    def _():
        pltpu.sync_copy(deg_sh.at[pl.ds(row0, R0)],
                        degout_hbm.at[cid, pl.ds(row0, R0)])

    @pl.when(last)
    def _():
        pltpu.sync_copy(deg_sh.at[pl.ds(row0, R15)],
                        degout_hbm.at[cid, pl.ds(row0, R15)])


RB = 2000


def _make_tc_layer(relu: bool):
    def body(p_ref, d_ref, w_ref, b_ref, o_ref):
        num = p_ref[0] + p_ref[1]
        deg = d_ref[0, :, :1] + d_ref[1, :, :1] + 1.0
        h = num / deg
        out = jnp.dot(h, w_ref[...], preferred_element_type=jnp.float32)
        out = out + b_ref[...]
        if relu:
            out = jnp.maximum(out, 0.0)
        o_ref[...] = out

    return pl.pallas_call(
        body,
        grid=(N // RB,),
        in_specs=[
            pl.BlockSpec((NC, RB, D), lambda i: (0, i, 0)),
            pl.BlockSpec((NC, RB, DEGW), lambda i: (0, i, 0)),
            pl.BlockSpec((D, D), lambda i: (0, 0)),
            pl.BlockSpec((1, D), lambda i: (0, 0)),
        ],
        out_specs=pl.BlockSpec((RB, D), lambda i: (i, 0)),
        out_shape=jax.ShapeDtypeStruct((N, D), jnp.float32),
    )


_tc_relu = _make_tc_layer(relu=True)
_tc_lin = _make_tc_layer(relu=False)


def kernel(g, features, W1, b1, W2, b2):
    src = g[0]
    dst = g[1]
    dst3 = dst.reshape(NW, NCHUNK, K)
    (degp,) = _sc_deg(dst3)
    (part1,) = _sc_agg(features, src, dst)
    h1 = _tc_relu(part1, degp, W1, b1.reshape(1, D))
    (part2,) = _sc_agg(h1, src, dst)
    out = _tc_lin(part2, degp, W2, b2.reshape(1, D))
    return out


# speedup vs baseline: 1.0533x; 1.0533x over previous
import functools

import jax
import jax.numpy as jnp
from jax import lax
from jax.experimental import pallas as pl
from jax.experimental.pallas import tpu as pltpu
from jax.experimental.pallas import tpu_sc as plsc

N = 10000
E = 320000
D = 128

NC = 2
NS = 16
NW = NC * NS
EPW = E // NW
K = 80
NCHUNK = EPW // K
R0 = 624
R15 = N - 15 * R0
DEGW = 128

_MESH = plsc.VectorSubcoreMesh(core_axis_name="c", subcore_axis_name="s")


def _make_sc_agg(with_deg: bool):
    out_type = [jax.ShapeDtypeStruct((NC, N, D), jnp.float32)]
    if with_deg:
        out_type.append(jax.ShapeDtypeStruct((NC, N, DEGW), jnp.float32))

    @functools.partial(
        pl.kernel, mesh=_MESH, out_type=out_type,
        scratch_types=[
            pltpu.VMEM_SHARED((N, D), jnp.float32),
            pltpu.VMEM((EPW,), jnp.int32),
            pltpu.VMEM((EPW,), jnp.int32),
            pltpu.VMEM((3, K, D), jnp.float32),
            pltpu.SemaphoreType.DMA,
            pltpu.SemaphoreType.DMA,
            pltpu.SemaphoreType.DMA,
        ])
    def body(x_hbm, src_hbm, dst_hbm, out_hbm, *rest):
        if with_deg:
            degout_hbm, acc_sh, sidx, didx, rows, sem0, sem1, sem2 = rest
        else:
            acc_sh, sidx, didx, rows, sem0, sem1, sem2 = rest
        cid = lax.axis_index("c")
        sid = lax.axis_index("s")
        wid = cid * NS + sid
        row0 = sid * R0
        last = sid == NS - 1
        sems = (sem0, sem1, sem2)

        def _gather(j, b):
            return pltpu.make_async_copy(
                x_hbm.at[sidx.at[pl.ds(j * K, K)]], rows.at[b], sems[b])

        def _fill_acc_rows(src_buf):
            @pl.when(jnp.logical_not(last))
            def _():
                for t in range(7):
                    pltpu.sync_copy(src_buf,
                                    acc_sh.at[pl.ds(row0 + t * K, K)])
                pltpu.sync_copy(src_buf.at[pl.ds(0, 64)],
                                acc_sh.at[pl.ds(row0 + 7 * K, 64)])

            @pl.when(last)
            def _():
                for t in range(8):
                    pltpu.sync_copy(src_buf,
                                    acc_sh.at[pl.ds(row0 + t * K, K)])

        def _write_out(dst_hbm_ref):
            @pl.when(jnp.logical_not(last))
            def _():
                pltpu.sync_copy(acc_sh.at[pl.ds(row0, R0)],
                                dst_hbm_ref.at[cid, pl.ds(row0, R0)])

            @pl.when(last)
            def _():
                pltpu.sync_copy(acc_sh.at[pl.ds(row0, R15)],
                                dst_hbm_ref.at[cid, pl.ds(row0, R15)])

        pltpu.sync_copy(src_hbm.at[pl.ds(wid * EPW, EPW)], sidx)
        pltpu.sync_copy(dst_hbm.at[pl.ds(wid * EPW, EPW)], didx)

        def zrow(r, carry):
            for j in range(D // 16):
                rows[0, r, pl.ds(j * 16, 16)] = jnp.zeros((16,),
                                                          jnp.float32)
            return carry
        lax.fori_loop(0, K, zrow, 0)

        if with_deg:
            def orow(r, carry):
                for j in range(D // 16):
                    rows[2, r, pl.ds(j * 16, 16)] = jnp.full(
                        (16,), 1.0, jnp.float32)
                return carry
            lax.fori_loop(0, K, orow, 0)
            _fill_acc_rows(rows.at[0])
            plsc.subcore_barrier()

            def fire(j, carry):
                pltpu.async_copy(rows.at[2],
                                 acc_sh.at[didx.at[pl.ds(j * K, K)]],
                                 sem0, add=True)
                return carry
            lax.fori_loop(0, NCHUNK, fire, 0)

            def drain(j, carry):
                pltpu.make_async_copy(
                    rows.at[2], acc_sh.at[didx.at[pl.ds(j * K, K)]],
                    sem0).wait()
                return carry
            lax.fori_loop(0, NCHUNK, drain, 0)
            plsc.subcore_barrier()
            _write_out(degout_hbm)
            plsc.subcore_barrier()

        @pl.when(jnp.logical_and(cid == 0, jnp.logical_not(last)))
        def _():
            pltpu.sync_copy(x_hbm.at[pl.ds(row0, R0)],
                            acc_sh.at[pl.ds(row0, R0)])

        @pl.when(jnp.logical_and(cid == 0, last))
        def _():
            pltpu.sync_copy(x_hbm.at[pl.ds(row0, R15)],
                            acc_sh.at[pl.ds(row0, R15)])

        @pl.when(cid != 0)
        def _():
            _fill_acc_rows(rows.at[0])

        plsc.subcore_barrier()

        _gather(0, 0).start()
        _gather(1, 1).start()

        def loop(t, carry):
            for o in range(3):
                j = 3 * t + o
                _gather(j + 2, (o + 2) % 3).start()
                _gather(j, o).wait()
                pltpu.sync_copy(rows.at[o],
                                acc_sh.at[didx.at[pl.ds(j * K, K)]],
                                add=True)
            return carry
        lax.fori_loop(0, (NCHUNK - 2) // 3, loop, 0)
        _gather(NCHUNK - 2, 0).wait()
        pltpu.sync_copy(rows.at[0],
                        acc_sh.at[didx.at[pl.ds((NCHUNK - 2) * K, K)]],
                        add=True)
        _gather(NCHUNK - 1, 1).wait()
        pltpu.sync_copy(rows.at[1],
                        acc_sh.at[didx.at[pl.ds((NCHUNK - 1) * K, K)]],
                        add=True)
        plsc.subcore_barrier()

        _write_out(out_hbm)

    return body


_sc_agg_deg = _make_sc_agg(with_deg=True)
_sc_agg = _make_sc_agg(with_deg=False)

RB = 2000


def _tc_layer1(part, degp, W, b):
    def body(p_ref, d_ref, w_ref, b_ref, o_ref, dc_ref):
        deg = d_ref[0, :, :1] + d_ref[1, :, :1] + 1.0
        h = (p_ref[0] + p_ref[1]) / deg
        out = jnp.dot(h, w_ref[...], preferred_element_type=jnp.float32)
        o_ref[...] = jnp.maximum(out + b_ref[...], 0.0)
        dc_ref[...] = deg

    return pl.pallas_call(
        body,
        grid=(N // RB,),
        in_specs=[
            pl.BlockSpec((NC, RB, D), lambda i: (0, i, 0)),
            pl.BlockSpec((NC, RB, DEGW), lambda i: (0, i, 0)),
            pl.BlockSpec((D, D), lambda i: (0, 0)),
            pl.BlockSpec((1, D), lambda i: (0, 0)),
        ],
        out_specs=[
            pl.BlockSpec((RB, D), lambda i: (i, 0)),
            pl.BlockSpec((RB, 1), lambda i: (i, 0)),
        ],
        out_shape=[
            jax.ShapeDtypeStruct((N, D), jnp.float32),
            jax.ShapeDtypeStruct((N, 1), jnp.float32),
        ],
    )(part, degp, W, b)


def _tc_layer2(part, degc, W, b):
    def body(p_ref, dc_ref, w_ref, b_ref, o_ref):
        h = (p_ref[0] + p_ref[1]) / dc_ref[...]
        out = jnp.dot(h, w_ref[...], preferred_element_type=jnp.float32)
        o_ref[...] = out + b_ref[...]

    return pl.pallas_call(
        body,
        grid=(N // RB,),
        in_specs=[
            pl.BlockSpec((NC, RB, D), lambda i: (0, i, 0)),
            pl.BlockSpec((RB, 1), lambda i: (i, 0)),
            pl.BlockSpec((D, D), lambda i: (0, 0)),
            pl.BlockSpec((1, D), lambda i: (0, 0)),
        ],
        out_specs=pl.BlockSpec((RB, D), lambda i: (i, 0)),
        out_shape=jax.ShapeDtypeStruct((N, D), jnp.float32),
    )(part, degc, W, b)


def kernel(g, features, W1, b1, W2, b2):
    src = g[0]
    dst = g[1]
    part1, degp = _sc_agg_deg(features, src, dst)
    h1, degc = _tc_layer1(part1, degp, W1, b1.reshape(1, D))
    (part2,) = _sc_agg(h1, src, dst)
    out = _tc_layer2(part2, degc, W2, b2.reshape(1, D))
    return out


# speedup vs baseline: 1.0606x; 1.0070x over previous
import functools

import jax
import jax.numpy as jnp
from jax import lax
from jax.experimental import pallas as pl
from jax.experimental.pallas import tpu as pltpu
from jax.experimental.pallas import tpu_sc as plsc

N = 10000
E = 320000
D = 128

NC = 2
NS = 16
NW = NC * NS
EPW = E // NW
K = 80
NCHUNK = EPW // K
R0 = 624
R15 = N - 15 * R0
DEGW = 128

_MESH = plsc.VectorSubcoreMesh(core_axis_name="c", subcore_axis_name="s")


def _make_sc_agg(with_deg: bool):
    out_type = [jax.ShapeDtypeStruct((NC, N, D), jnp.float32)]
    if with_deg:
        out_type.append(jax.ShapeDtypeStruct((NC, N, DEGW), jnp.float32))

    @functools.partial(
        pl.kernel, mesh=_MESH, out_type=out_type,
        scratch_types=[
            pltpu.VMEM_SHARED((N, D), jnp.float32),
            pltpu.VMEM((EPW,), jnp.int32),
            pltpu.VMEM((EPW,), jnp.int32),
            pltpu.VMEM((3, K, D), jnp.float32),
            pltpu.SemaphoreType.DMA,
            pltpu.SemaphoreType.DMA,
            pltpu.SemaphoreType.DMA,
        ])
    def body(x_hbm, src_hbm, dst_hbm, out_hbm, *rest):
        if with_deg:
            degout_hbm, acc_sh, sidx, didx, rows, sem0, sem1, sem2 = rest
        else:
            acc_sh, sidx, didx, rows, sem0, sem1, sem2 = rest
        cid = lax.axis_index("c")
        sid = lax.axis_index("s")
        wid = cid * NS + sid
        row0 = sid * R0
        last = sid == NS - 1
        sems = (sem0, sem1, sem2)

        def _gather(j, b):
            return pltpu.make_async_copy(
                x_hbm.at[sidx.at[pl.ds(j * K, K)]], rows.at[b], sems[b])

        def _fill_acc_rows(src_buf):
            @pl.when(jnp.logical_not(last))
            def _():
                for t in range(7):
                    pltpu.sync_copy(src_buf,
                                    acc_sh.at[pl.ds(row0 + t * K, K)])
                pltpu.sync_copy(src_buf.at[pl.ds(0, 64)],
                                acc_sh.at[pl.ds(row0 + 7 * K, 64)])

            @pl.when(last)
            def _():
                for t in range(8):
                    pltpu.sync_copy(src_buf,
                                    acc_sh.at[pl.ds(row0 + t * K, K)])

        def _write_out(dst_hbm_ref):
            @pl.when(jnp.logical_not(last))
            def _():
                pltpu.sync_copy(acc_sh.at[pl.ds(row0, R0)],
                                dst_hbm_ref.at[cid, pl.ds(row0, R0)])

            @pl.when(last)
            def _():
                pltpu.sync_copy(acc_sh.at[pl.ds(row0, R15)],
                                dst_hbm_ref.at[cid, pl.ds(row0, R15)])

        pltpu.sync_copy(src_hbm.at[pl.ds(wid * EPW, EPW)], sidx)
        pltpu.sync_copy(dst_hbm.at[pl.ds(wid * EPW, EPW)], didx)

        zb = 0 if with_deg else 2
        if not with_deg:
            _gather(0, 0).start()
            _gather(1, 1).start()

        def zrow(r, carry):
            for j in range(D // 16):
                rows[zb, r, pl.ds(j * 16, 16)] = jnp.zeros((16,),
                                                           jnp.float32)
            return carry
        lax.fori_loop(0, K, zrow, 0)

        if with_deg:
            def orow(r, carry):
                for j in range(D // 16):
                    rows[2, r, pl.ds(j * 16, 16)] = jnp.full(
                        (16,), 1.0, jnp.float32)
                return carry
            lax.fori_loop(0, K, orow, 0)
            _fill_acc_rows(rows.at[0])
            plsc.subcore_barrier()

            def fire(j, carry):
                pltpu.async_copy(rows.at[2],
                                 acc_sh.at[didx.at[pl.ds(j * K, K)]],
                                 sem0, add=True)
                return carry
            lax.fori_loop(0, NCHUNK, fire, 0)

            def drain(j, carry):
                pltpu.make_async_copy(
                    rows.at[2], acc_sh.at[didx.at[pl.ds(j * K, K)]],
                    sem0).wait()
                return carry
            lax.fori_loop(0, NCHUNK, drain, 0)
            plsc.subcore_barrier()
            _write_out(degout_hbm)
            plsc.subcore_barrier()

        @pl.when(jnp.logical_and(cid == 0, jnp.logical_not(last)))
        def _():
            pltpu.sync_copy(x_hbm.at[pl.ds(row0, R0)],
                            acc_sh.at[pl.ds(row0, R0)])

        @pl.when(jnp.logical_and(cid == 0, last))
        def _():
            pltpu.sync_copy(x_hbm.at[pl.ds(row0, R15)],
                            acc_sh.at[pl.ds(row0, R15)])

        @pl.when(cid != 0)
        def _():
            _fill_acc_rows(rows.at[zb])

        plsc.subcore_barrier()

        if with_deg:
            _gather(0, 0).start()
            _gather(1, 1).start()

        def loop(t, carry):
            for o in range(3):
                j = 3 * t + o
                _gather(j + 2, (o + 2) % 3).start()
                _gather(j, o).wait()
                pltpu.sync_copy(rows.at[o],
                                acc_sh.at[didx.at[pl.ds(j * K, K)]],
                                add=True)
            return carry
        lax.fori_loop(0, (NCHUNK - 2) // 3, loop, 0)
        _gather(NCHUNK - 2, 0).wait()
        pltpu.sync_copy(rows.at[0],
                        acc_sh.at[didx.at[pl.ds((NCHUNK - 2) * K, K)]],
                        add=True)
        _gather(NCHUNK - 1, 1).wait()
        pltpu.sync_copy(rows.at[1],
                        acc_sh.at[didx.at[pl.ds((NCHUNK - 1) * K, K)]],
                        add=True)
        plsc.subcore_barrier()

        _write_out(out_hbm)

    return body


_sc_agg_deg = _make_sc_agg(with_deg=True)
_sc_agg = _make_sc_agg(with_deg=False)

RB = 2000


def _tc_layer1(part, degp, W, b):
    def body(p_ref, d_ref, w_ref, b_ref, o_ref, dc_ref):
        deg = d_ref[0, :, :1] + d_ref[1, :, :1] + 1.0
        h = (p_ref[0] + p_ref[1]) / deg
        out = jnp.dot(h, w_ref[...], preferred_element_type=jnp.float32)
        o_ref[...] = jnp.maximum(out + b_ref[...], 0.0)
        dc_ref[...] = deg

    return pl.pallas_call(
        body,
        grid=(N // RB,),
        in_specs=[
            pl.BlockSpec((NC, RB, D), lambda i: (0, i, 0)),
            pl.BlockSpec((NC, RB, DEGW), lambda i: (0, i, 0)),
            pl.BlockSpec((D, D), lambda i: (0, 0)),
            pl.BlockSpec((1, D), lambda i: (0, 0)),
        ],
        out_specs=[
            pl.BlockSpec((RB, D), lambda i: (i, 0)),
            pl.BlockSpec((RB, 1), lambda i: (i, 0)),
        ],
        out_shape=[
            jax.ShapeDtypeStruct((N, D), jnp.float32),
            jax.ShapeDtypeStruct((N, 1), jnp.float32),
        ],
    )(part, degp, W, b)


def _tc_layer2(part, degc, W, b):
    def body(p_ref, dc_ref, w_ref, b_ref, o_ref):
        h = (p_ref[0] + p_ref[1]) / dc_ref[...]
        out = jnp.dot(h, w_ref[...], preferred_element_type=jnp.float32)
        o_ref[...] = out + b_ref[...]

    return pl.pallas_call(
        body,
        grid=(N // RB,),
        in_specs=[
            pl.BlockSpec((NC, RB, D), lambda i: (0, i, 0)),
            pl.BlockSpec((RB, 1), lambda i: (i, 0)),
            pl.BlockSpec((D, D), lambda i: (0, 0)),
            pl.BlockSpec((1, D), lambda i: (0, 0)),
        ],
        out_specs=pl.BlockSpec((RB, D), lambda i: (i, 0)),
        out_shape=jax.ShapeDtypeStruct((N, D), jnp.float32),
    )(part, degc, W, b)


def kernel(g, features, W1, b1, W2, b2):
    src = g[0]
    dst = g[1]
    part1, degp = _sc_agg_deg(features, src, dst)
    h1, degc = _tc_layer1(part1, degp, W1, b1.reshape(1, D))
    (part2,) = _sc_agg(h1, src, dst)
    out = _tc_layer2(part2, degc, W2, b2.reshape(1, D))
    return out
